# Initial kernel scaffold; baseline (speedup 1.0000x reference)
import jax
import jax.numpy as jnp
from jax.experimental import pallas as pl


def kernel(x, edge_index, batch, data2, emb, W0, as0, ad0, b0, W1, as1, ad1, b1, Wl, bl):
    raise NotImplementedError("write your pallas kernel here")


# speedup vs baseline: 1.4610x; 1.4610x over previous
import jax
import jax.numpy as jnp
from jax import lax
from jax.experimental import pallas as pl
from jax.experimental.pallas import tpu as pltpu

N = 10000
D = 128
G = 64
ROW_BLK = 1000
N_BLKS = N // ROW_BLK


def _leaky_relu(v, slope=0.2):
    return jnp.where(v >= 0, v, slope * v)


def _gat_layer_xla(h, src, dst, W, a_src, a_dst, b, num_nodes):
    h = h @ W.T
    e = _leaky_relu((h * a_src).sum(-1)[src] + (h * a_dst).sum(-1)[dst])
    ee = jnp.exp(e)
    s = jax.ops.segment_sum(ee, dst, num_segments=num_nodes)
    acc = jax.ops.segment_sum(ee[:, None] * h[src], dst, num_segments=num_nodes)
    return acc / (s[:, None] + 1e-16) + b


def _pool_linear_kernel(batch_ref, h_ref, wl_ref, bl_ref, out_ref, acc_ref, cnt_ref):
    i = pl.program_id(0)

    @pl.when(i == 0)
    def _():
        acc_ref[...] = jnp.zeros_like(acc_ref)
        cnt_ref[...] = jnp.zeros_like(cnt_ref)

    seg = batch_ref[...]
    gid = lax.broadcasted_iota(jnp.int32, (ROW_BLK, G), 1)
    onehot = (seg == gid).astype(jnp.float32)
    h = h_ref[...]
    acc_ref[...] += lax.dot_general(
        onehot, h, (((0,), (0,)), ((), ())), preferred_element_type=jnp.float32)
    cnt_ref[...] += lax.dot_general(
        onehot, jnp.ones_like(h), (((0,), (0,)), ((), ())),
        preferred_element_type=jnp.float32)

    @pl.when(i == N_BLKS - 1)
    def _():
        pooled = acc_ref[...] / jnp.maximum(cnt_ref[...], 1.0)
        out_ref[...] = jnp.tanh(
            lax.dot_general(pooled, wl_ref[...], (((1,), (1,)), ((), ())),
                            preferred_element_type=jnp.float32) + bl_ref[...])


def _pool_linear(batch, h, Wl, bl):
    return pl.pallas_call(
        _pool_linear_kernel,
        grid=(N_BLKS,),
        in_specs=[
            pl.BlockSpec((ROW_BLK, 1), lambda i: (i, 0)),
            pl.BlockSpec((ROW_BLK, D), lambda i: (i, 0)),
            pl.BlockSpec((D, D), lambda i: (0, 0)),
            pl.BlockSpec((1, D), lambda i: (0, 0)),
        ],
        out_specs=pl.BlockSpec((G, D), lambda i: (0, 0)),
        out_shape=jax.ShapeDtypeStruct((G, D), jnp.float32),
        scratch_shapes=[
            pltpu.VMEM((G, D), jnp.float32),
            pltpu.VMEM((G, D), jnp.float32),
        ],
    )(batch.reshape(N, 1).astype(jnp.int32), h, Wl, bl.reshape(1, D))


def kernel(x, edge_index, batch, data2, emb, W0, as0, ad0, b0, W1, as1, ad1, b1, Wl, bl):
    del data2
    h = jnp.take(emb, x, axis=0)
    loop = jnp.arange(N)
    src = jnp.concatenate([edge_index[0], loop])
    dst = jnp.concatenate([edge_index[1], loop])
    for (W, a_s, a_d, b) in ((W0, as0, ad0, b0), (W1, as1, ad1, b1)):
        h = jnp.tanh(_gat_layer_xla(h, src, dst, W, a_s, a_d, b, N))
    return _pool_linear(batch, h, Wl, bl)


# speedup vs baseline: 21.7048x; 14.8560x over previous
import dataclasses
import functools

import jax
import jax.numpy as jnp
from jax import lax
from jax.experimental import pallas as pl
from jax.experimental.pallas import tpu as pltpu
from jax.experimental.pallas import tpu_sc as plsc

N = 10000
D = 128
G = 64
E = 320000

NC = 2
NS = 16
NW = NC * NS

NPAD = 10240
ROW_BLK = 1280
N_BLKS = NPAD // ROW_BLK
XPT = NPAD // NW

CH = 128
NCHUNK = 81
EPAD = NW * NCHUNK * CH
EXTRA = EPAD - E - N

DEN_BASE = 10112
DEN_ROWS = 80
APAD = 10240
RPT = APAD // NS

_vector_mesh = plsc.VectorSubcoreMesh(core_axis_name="c", subcore_axis_name="s")

_sc_params = pltpu.CompilerParams()
if "needs_layout_passes" in pltpu.CompilerParams.__dataclass_fields__:
    _sc_params = dataclasses.replace(_sc_params, needs_layout_passes=False)


def _sc_gather_rows(table, idx):
    @functools.partial(
        pl.kernel,
        mesh=_vector_mesh,
        out_type=jax.ShapeDtypeStruct((NPAD, D), jnp.float32),
        scratch_types=[
            pltpu.VMEM((XPT,), jnp.int32),
            pltpu.VMEM((XPT, D), jnp.float32),
            pltpu.SemaphoreType.DMA,
        ],
    )
    def k(table_hbm, idx_hbm, out_hbm, idx_v, rows_v, sem):
        wid = lax.axis_index("s") * NC + lax.axis_index("c")
        base = wid * XPT
        pltpu.sync_copy(idx_hbm.at[pl.ds(base, XPT)], idx_v)
        pltpu.async_copy(table_hbm.at[idx_v], rows_v, sem).wait()
        pltpu.sync_copy(rows_v, out_hbm.at[pl.ds(base, XPT)])

    return k(table, idx)


def _sc_edge_kernel(hw_hbm, asn_hbm, adn_hbm, sd_hbm, out_hbm,
                    asn_t, adn_t, sd_t, ee_t, rows_t, den_t,
                    den_rows_t, acc, sem):
    c = lax.axis_index("c")
    s = lax.axis_index("s")
    wid = s * NC + c

    pltpu.sync_copy(asn_hbm, asn_t)
    pltpu.sync_copy(adn_hbm, adn_t)

    zv = jnp.zeros((16,), jnp.float32)
    i16 = lax.broadcasted_iota(jnp.int32, (16,), 0)

    @pl.loop(0, CH)
    def _(r):
        @pl.loop(0, D, step=16)
        def _(k2):
            rows_t[r, pl.ds(k2, 16)] = zv

    @pl.loop(0, DEN_ROWS)
    def _(r):
        @pl.loop(0, D, step=16)
        def _(k2):
            den_t[r, pl.ds(k2, 16)] = zv

    @pl.loop(0, DEN_ROWS, step=16)
    def _(k2):
        den_rows_t[pl.ds(k2, 16)] = DEN_BASE + k2 + i16

    @pl.loop(0, RPT, step=CH)
    def _(q):
        pltpu.sync_copy(rows_t, acc.at[pl.ds(s * RPT + q, CH)])

    plsc.subcore_barrier()

    @pl.loop(0, NCHUNK)
    def _(j):
        pltpu.sync_copy(sd_hbm.at[wid, j], sd_t)

        @pl.loop(0, CH, step=16)
        def _(g):
            sv = sd_t[0, pl.ds(g, 16)]
            dv = sd_t[1, pl.ds(g, 16)]
            e = plsc.load_gather(asn_t, [sv]) + plsc.load_gather(adn_t, [dv])
            e = jnp.where(e >= 0.0, e, 0.2 * e)
            ee = jnp.exp(e)
            ee_t[pl.ds(g, 16)] = ee
            plsc.addupdate_scatter(
                den_t, [lax.shift_right_logical(dv, 7),
                        lax.bitwise_and(dv, 127)], ee)

        pltpu.async_copy(hw_hbm.at[sd_t.at[0]], rows_t, sem).wait()

        @pl.loop(0, CH)
        def _(r):
            eev = plsc.load_gather(ee_t, [jnp.zeros((16,), jnp.int32) + r])
            for k2 in range(D // 16):
                rows_t[r, pl.ds(k2 * 16, 16)] = (
                    rows_t[r, pl.ds(k2 * 16, 16)] * eev)

        pltpu.sync_copy(rows_t, acc.at[sd_t.at[1]], add=True)

    pltpu.sync_copy(den_t, acc.at[den_rows_t], add=True)

    plsc.subcore_barrier()

    @pl.loop(0, RPT, step=CH)
    def _(q):
        r0 = s * RPT + q
        pltpu.sync_copy(acc.at[pl.ds(r0, CH)], out_hbm.at[c, pl.ds(r0, CH)])


def _sc_edge(hw, asn, adn, sd):
    @functools.partial(
        pl.kernel,
        mesh=_vector_mesh,
        compiler_params=_sc_params,
        out_type=jax.ShapeDtypeStruct((NC, APAD, D), jnp.float32),
        scratch_types=[
            pltpu.VMEM((NPAD,), jnp.float32),
            pltpu.VMEM((NPAD,), jnp.float32),
            pltpu.VMEM((2, CH), jnp.int32),
            pltpu.VMEM((CH,), jnp.float32),
            pltpu.VMEM((CH, D), jnp.float32),
            pltpu.VMEM((DEN_ROWS, D), jnp.float32),
            pltpu.VMEM((DEN_ROWS,), jnp.int32),
            pltpu.VMEM_SHARED((APAD, D), jnp.float32),
            pltpu.SemaphoreType.DMA,
        ],
    )
    def k(*refs):
        _sc_edge_kernel(*refs)

    return k(hw, asn, adn, sd)


def _row_mask(i):
    row = i * ROW_BLK + lax.broadcasted_iota(jnp.int32, (ROW_BLK, 1), 0)
    return row < N


def _project_kernel(h_ref, w_ref, asd_ref, hw_ref, asnadn_ref):
    hw = lax.dot_general(h_ref[...], w_ref[...], (((1,), (1,)), ((), ())),
                         preferred_element_type=jnp.float32)
    hw_ref[...] = hw
    asnadn_ref[...] = lax.dot_general(hw, asd_ref[...], (((1,), (1,)), ((), ())),
                                      preferred_element_type=jnp.float32)


def _tc_project(h, W, a_s, a_d):
    asd = jnp.stack([a_s, a_d])
    return pl.pallas_call(
        _project_kernel,
        grid=(N_BLKS,),
        in_specs=[
            pl.BlockSpec((ROW_BLK, D), lambda i: (i, 0)),
            pl.BlockSpec((D, D), lambda i: (0, 0)),
            pl.BlockSpec((2, D), lambda i: (0, 0)),
        ],
        out_specs=[
            pl.BlockSpec((ROW_BLK, D), lambda i: (i, 0)),
            pl.BlockSpec((ROW_BLK, 2), lambda i: (i, 0)),
        ],
        out_shape=[
            jax.ShapeDtypeStruct((NPAD, D), jnp.float32),
            jax.ShapeDtypeStruct((NPAD, 2), jnp.float32),
        ],
    )(h, W, asd)


def _combine(i, acc_arr, den_arr, b_ref):
    num = acc_arr[0] + acc_arr[1]
    den = den_arr[0] + den_arr[1]
    h = jnp.tanh(num / (den + 1e-16) + b_ref[...])
    return jnp.where(_row_mask(i), h, 0.0)


def _combine_project_kernel(acc_ref, den_ref, b_ref, w_ref, asd_ref,
                            hw_ref, asnadn_ref):
    h = _combine(pl.program_id(0), acc_ref[...], den_ref[...], b_ref)
    hw = lax.dot_general(h, w_ref[...], (((1,), (1,)), ((), ())),
                         preferred_element_type=jnp.float32)
    hw_ref[...] = hw
    asnadn_ref[...] = lax.dot_general(hw, asd_ref[...], (((1,), (1,)), ((), ())),
                                      preferred_element_type=jnp.float32)


def _tc_combine_project(acc, den, b, W, a_s, a_d):
    asd = jnp.stack([a_s, a_d])
    return pl.pallas_call(
        _combine_project_kernel,
        grid=(N_BLKS,),
        in_specs=[
            pl.BlockSpec((NC, ROW_BLK, D), lambda i: (0, i, 0)),
            pl.BlockSpec((NC, ROW_BLK, 1), lambda i: (0, i, 0)),
            pl.BlockSpec((1, D), lambda i: (0, 0)),
            pl.BlockSpec((D, D), lambda i: (0, 0)),
            pl.BlockSpec((2, D), lambda i: (0, 0)),
        ],
        out_specs=[
            pl.BlockSpec((ROW_BLK, D), lambda i: (i, 0)),
            pl.BlockSpec((ROW_BLK, 2), lambda i: (i, 0)),
        ],
        out_shape=[
            jax.ShapeDtypeStruct((NPAD, D), jnp.float32),
            jax.ShapeDtypeStruct((NPAD, 2), jnp.float32),
        ],
    )(acc, den, b.reshape(1, D), W, asd)


def _combine_pool_kernel(acc_ref, den_ref, b_ref, batch_ref, wl_ref, bl_ref,
                         out_ref, accp_ref, cnt_ref):
    i = pl.program_id(0)

    @pl.when(i == 0)
    def _():
        accp_ref[...] = jnp.zeros_like(accp_ref)
        cnt_ref[...] = jnp.zeros_like(cnt_ref)

    h = _combine(i, acc_ref[...], den_ref[...], b_ref)
    seg = batch_ref[...]
    gid = lax.broadcasted_iota(jnp.int32, (ROW_BLK, G), 1)
    onehot = (seg == gid).astype(jnp.float32)
    accp_ref[...] += lax.dot_general(onehot, h, (((0,), (0,)), ((), ())),
                                     preferred_element_type=jnp.float32)
    cnt_ref[...] += lax.dot_general(onehot, jnp.ones_like(h),
                                    (((0,), (0,)), ((), ())),
                                    preferred_element_type=jnp.float32)

    @pl.when(i == N_BLKS - 1)
    def _():
        pooled = accp_ref[...] / jnp.maximum(cnt_ref[...], 1.0)
        out_ref[...] = jnp.tanh(
            lax.dot_general(pooled, wl_ref[...], (((1,), (1,)), ((), ())),
                            preferred_element_type=jnp.float32) + bl_ref[...])


def _tc_combine_pool(acc, den, b, batch_pad, Wl, bl):
    return pl.pallas_call(
        _combine_pool_kernel,
        grid=(N_BLKS,),
        in_specs=[
            pl.BlockSpec((NC, ROW_BLK, D), lambda i: (0, i, 0)),
            pl.BlockSpec((NC, ROW_BLK, 1), lambda i: (0, i, 0)),
            pl.BlockSpec((1, D), lambda i: (0, 0)),
            pl.BlockSpec((ROW_BLK, 1), lambda i: (i, 0)),
            pl.BlockSpec((D, D), lambda i: (0, 0)),
            pl.BlockSpec((1, D), lambda i: (0, 0)),
        ],
        out_specs=pl.BlockSpec((G, D), lambda i: (0, 0)),
        out_shape=jax.ShapeDtypeStruct((G, D), jnp.float32),
        scratch_shapes=[
            pltpu.VMEM((G, D), jnp.float32),
            pltpu.VMEM((G, D), jnp.float32),
        ],
    )(acc, den, b.reshape(1, D), batch_pad, Wl, bl.reshape(1, D))


def _split_acc(out):
    den = out[:, DEN_BASE:DEN_BASE + DEN_ROWS, :].reshape(NC, DEN_ROWS * D, 1)
    return out, den[:, :NPAD]


def kernel(x, edge_index, batch, data2, emb, W0, as0, ad0, b0, W1, as1, ad1,
           b1, Wl, bl):
    del data2
    x32 = x.astype(jnp.int32)
    xpad = jnp.concatenate([x32, jnp.zeros((NPAD - N,), jnp.int32)])
    h0 = _sc_gather_rows(emb, xpad)

    loop = jnp.arange(N, dtype=jnp.int32)
    src = jnp.concatenate(
        [edge_index[0].astype(jnp.int32), loop,
         jnp.zeros((EXTRA,), jnp.int32)]).reshape(NW, NCHUNK, CH)
    dst = jnp.concatenate(
        [edge_index[1].astype(jnp.int32), loop,
         jnp.full((EXTRA,), N, jnp.int32)]).reshape(NW, NCHUNK, CH)
    sd = jnp.stack([src, dst], axis=2)

    hw, asnadn = _tc_project(h0, W0, as0, ad0)
    acc0, den0 = _split_acc(_sc_edge(hw, asnadn[:, 0], asnadn[:, 1], sd))

    hw1, asnadn1 = _tc_combine_project(acc0, den0, b0, W1, as1, ad1)
    acc1, den1 = _split_acc(_sc_edge(hw1, asnadn1[:, 0], asnadn1[:, 1], sd))

    batch_pad = jnp.concatenate(
        [batch.astype(jnp.int32), jnp.full((NPAD - N,), G, jnp.int32)])
    return _tc_combine_pool(acc1, den1, b1, batch_pad.reshape(NPAD, 1), Wl, bl)


# speedup vs baseline: 23.7465x; 1.0941x over previous
import dataclasses
import functools

import jax
import jax.numpy as jnp
from jax import lax
from jax.experimental import pallas as pl
from jax.experimental.pallas import tpu as pltpu
from jax.experimental.pallas import tpu_sc as plsc

N = 10000
D = 128
G = 64
E = 320000

NC = 2
NS = 16
NW = NC * NS

NPAD = 10240
ROW_BLK = 1280
N_BLKS = NPAD // ROW_BLK
XPT = NPAD // NW

CH = 64
NCHUNK = 162
EPAD = NW * NCHUNK * CH
EXTRA = EPAD - E - N

DEN_BASE = 10112
DEN_ROWS = 80
APAD = 10240
RPT = APAD // NS

_vector_mesh = plsc.VectorSubcoreMesh(core_axis_name="c", subcore_axis_name="s")

_sc_params = pltpu.CompilerParams()
if "needs_layout_passes" in pltpu.CompilerParams.__dataclass_fields__:
    _sc_params = dataclasses.replace(_sc_params, needs_layout_passes=False)


def _sc_gather_rows(table, idx):
    @functools.partial(
        pl.kernel,
        mesh=_vector_mesh,
        out_type=jax.ShapeDtypeStruct((NPAD, D), jnp.float32),
        scratch_types=[
            pltpu.VMEM((XPT,), jnp.int32),
            pltpu.VMEM((XPT, D), jnp.float32),
            pltpu.SemaphoreType.DMA,
        ],
    )
    def k(table_hbm, idx_hbm, out_hbm, idx_v, rows_v, sem):
        wid = lax.axis_index("s") * NC + lax.axis_index("c")
        base = wid * XPT
        pltpu.sync_copy(idx_hbm.at[pl.ds(base, XPT)], idx_v)
        pltpu.async_copy(table_hbm.at[idx_v], rows_v, sem).wait()
        pltpu.sync_copy(rows_v, out_hbm.at[pl.ds(base, XPT)])

    return k(table, idx)


def _sc_edge_kernel(hw_hbm, asn_hbm, adn_hbm, sd_hbm, out_hbm,
                    asn_t, adn_t, sd2_t, dsc_t, ee_t, rows2_t, den_t,
                    den_rows_t, acc, gsem0, gsem1, ssem0, ssem1, dsem0, dsem1):
    gsem = (gsem0, gsem1)
    ssem = (ssem0, ssem1)
    dsem = (dsem0, dsem1)
    c = lax.axis_index("c")
    s = lax.axis_index("s")
    wid = s * NC + c

    pltpu.sync_copy(asn_hbm, asn_t)
    pltpu.sync_copy(adn_hbm, adn_t)

    zv = jnp.zeros((16,), jnp.float32)
    zvi = jnp.zeros((16,), jnp.int32)
    i16 = lax.broadcasted_iota(jnp.int32, (16,), 0)

    @pl.loop(0, CH)
    def _(r):
        @pl.loop(0, D, step=16)
        def _(k2):
            rows2_t[0, r, pl.ds(k2, 16)] = zv
            rows2_t[1, r, pl.ds(k2, 16)] = zv

    @pl.loop(0, CH, step=16)
    def _(g):
        dsc_t[0, pl.ds(g, 16)] = zvi
        dsc_t[1, pl.ds(g, 16)] = zvi

    @pl.loop(0, DEN_ROWS)
    def _(r):
        @pl.loop(0, D, step=16)
        def _(k2):
            den_t[r, pl.ds(k2, 16)] = zv

    @pl.loop(0, DEN_ROWS, step=16)
    def _(k2):
        den_rows_t[pl.ds(k2, 16)] = DEN_BASE + k2 + i16

    @pl.loop(0, RPT, step=CH)
    def _(q):
        pltpu.sync_copy(rows2_t.at[0], acc.at[pl.ds(s * RPT + q, CH)])

    plsc.subcore_barrier()

    pltpu.async_copy(rows2_t.at[0], acc.at[dsc_t.at[0]], ssem[0], add=True)
    pltpu.async_copy(rows2_t.at[1], acc.at[dsc_t.at[1]], ssem[1], add=True)
    pltpu.async_copy(sd_hbm.at[wid, 0], sd2_t.at[0], dsem[0])
    pltpu.async_copy(sd_hbm.at[wid, 1], sd2_t.at[1], dsem[1])

    @pl.loop(0, NCHUNK, step=2)
    def _(j):
        for b in range(2):
            i = j + b
            pltpu.make_async_copy(hw_hbm.at[pl.ds(0, CH)], rows2_t.at[b],
                                  ssem[b]).wait()
            pltpu.make_async_copy(sd_hbm.at[wid, 0], sd2_t.at[b],
                                  dsem[b]).wait()
            gcopy = pltpu.async_copy(hw_hbm.at[sd2_t.at[b, 0]],
                                     rows2_t.at[b], gsem[b])

            @pl.loop(0, CH, step=16)
            def _(g):
                sv = sd2_t[b, 0, pl.ds(g, 16)]
                dv = sd2_t[b, 1, pl.ds(g, 16)]
                e = (plsc.load_gather(asn_t, [sv])
                     + plsc.load_gather(adn_t, [dv]))
                e = jnp.where(e >= 0.0, e, 0.2 * e)
                ee = jnp.exp(e)
                ee_t[pl.ds(g, 16)] = ee
                plsc.addupdate_scatter(
                    den_t, [lax.shift_right_logical(dv, 7),
                            lax.bitwise_and(dv, 127)], ee)
                dsc_t[b, pl.ds(g, 16)] = dv

            gcopy.wait()
            pltpu.async_copy(sd_hbm.at[wid, jnp.minimum(i + 2, NCHUNK - 1)],
                             sd2_t.at[b], dsem[b])

            @pl.loop(0, CH)
            def _(r):
                eev = plsc.load_gather(ee_t, [zvi + r])
                for k2 in range(D // 16):
                    rows2_t[b, r, pl.ds(k2 * 16, 16)] = (
                        rows2_t[b, r, pl.ds(k2 * 16, 16)] * eev)

            pltpu.async_copy(rows2_t.at[b], acc.at[dsc_t.at[b]], ssem[b],
                             add=True)

    for b in range(2):
        pltpu.make_async_copy(hw_hbm.at[pl.ds(0, CH)], rows2_t.at[b],
                              ssem[b]).wait()
        pltpu.make_async_copy(sd_hbm.at[wid, 0], sd2_t.at[b], dsem[b]).wait()

    pltpu.sync_copy(den_t, acc.at[den_rows_t], add=True)

    plsc.subcore_barrier()

    @pl.loop(0, RPT, step=2 * CH)
    def _(q):
        r0 = s * RPT + q
        pltpu.sync_copy(acc.at[pl.ds(r0, 2 * CH)],
                        out_hbm.at[c, pl.ds(r0, 2 * CH)])


def _sc_edge(hw, asn, adn, sd):
    @functools.partial(
        pl.kernel,
        mesh=_vector_mesh,
        compiler_params=_sc_params,
        out_type=jax.ShapeDtypeStruct((NC, APAD, D), jnp.float32),
        scratch_types=[
            pltpu.VMEM((NPAD,), jnp.float32),
            pltpu.VMEM((NPAD,), jnp.float32),
            pltpu.VMEM((2, 2, CH), jnp.int32),
            pltpu.VMEM((2, CH), jnp.int32),
            pltpu.VMEM((CH,), jnp.float32),
            pltpu.VMEM((2, CH, D), jnp.float32),
            pltpu.VMEM((DEN_ROWS, D), jnp.float32),
            pltpu.VMEM((DEN_ROWS,), jnp.int32),
            pltpu.VMEM_SHARED((APAD, D), jnp.float32),
            pltpu.SemaphoreType.DMA,
            pltpu.SemaphoreType.DMA,
            pltpu.SemaphoreType.DMA,
            pltpu.SemaphoreType.DMA,
            pltpu.SemaphoreType.DMA,
            pltpu.SemaphoreType.DMA,
        ],
    )
    def k(*refs):
        _sc_edge_kernel(*refs)

    return k(hw, asn, adn, sd)


def _row_mask(i):
    row = i * ROW_BLK + lax.broadcasted_iota(jnp.int32, (ROW_BLK, 1), 0)
    return row < N


def _project_kernel(h_ref, w_ref, asd_ref, hw_ref, asnadn_ref):
    hw = lax.dot_general(h_ref[...], w_ref[...], (((1,), (1,)), ((), ())),
                         preferred_element_type=jnp.float32)
    hw_ref[...] = hw
    asnadn_ref[...] = lax.dot_general(hw, asd_ref[...], (((1,), (1,)), ((), ())),
                                      preferred_element_type=jnp.float32)


def _tc_project(h, W, a_s, a_d):
    asd = jnp.stack([a_s, a_d])
    return pl.pallas_call(
        _project_kernel,
        grid=(N_BLKS,),
        in_specs=[
            pl.BlockSpec((ROW_BLK, D), lambda i: (i, 0)),
            pl.BlockSpec((D, D), lambda i: (0, 0)),
            pl.BlockSpec((2, D), lambda i: (0, 0)),
        ],
        out_specs=[
            pl.BlockSpec((ROW_BLK, D), lambda i: (i, 0)),
            pl.BlockSpec((ROW_BLK, 2), lambda i: (i, 0)),
        ],
        out_shape=[
            jax.ShapeDtypeStruct((NPAD, D), jnp.float32),
            jax.ShapeDtypeStruct((NPAD, 2), jnp.float32),
        ],
    )(h, W, asd)


def _combine(i, acc_arr, den_arr, b_ref):
    num = acc_arr[0] + acc_arr[1]
    den = den_arr[0] + den_arr[1]
    h = jnp.tanh(num / (den + 1e-16) + b_ref[...])
    return jnp.where(_row_mask(i), h, 0.0)


def _combine_project_kernel(acc_ref, den_ref, b_ref, w_ref, asd_ref,
                            hw_ref, asnadn_ref):
    h = _combine(pl.program_id(0), acc_ref[...], den_ref[...], b_ref)
    hw = lax.dot_general(h, w_ref[...], (((1,), (1,)), ((), ())),
                         preferred_element_type=jnp.float32)
    hw_ref[...] = hw
    asnadn_ref[...] = lax.dot_general(hw, asd_ref[...], (((1,), (1,)), ((), ())),
                                      preferred_element_type=jnp.float32)


def _tc_combine_project(acc, den, b, W, a_s, a_d):
    asd = jnp.stack([a_s, a_d])
    return pl.pallas_call(
        _combine_project_kernel,
        grid=(N_BLKS,),
        in_specs=[
            pl.BlockSpec((NC, ROW_BLK, D), lambda i: (0, i, 0)),
            pl.BlockSpec((NC, ROW_BLK, 1), lambda i: (0, i, 0)),
            pl.BlockSpec((1, D), lambda i: (0, 0)),
            pl.BlockSpec((D, D), lambda i: (0, 0)),
            pl.BlockSpec((2, D), lambda i: (0, 0)),
        ],
        out_specs=[
            pl.BlockSpec((ROW_BLK, D), lambda i: (i, 0)),
            pl.BlockSpec((ROW_BLK, 2), lambda i: (i, 0)),
        ],
        out_shape=[
            jax.ShapeDtypeStruct((NPAD, D), jnp.float32),
            jax.ShapeDtypeStruct((NPAD, 2), jnp.float32),
        ],
    )(acc, den, b.reshape(1, D), W, asd)


def _combine_pool_kernel(acc_ref, den_ref, b_ref, batch_ref, wl_ref, bl_ref,
                         out_ref, accp_ref, cnt_ref):
    i = pl.program_id(0)

    @pl.when(i == 0)
    def _():
        accp_ref[...] = jnp.zeros_like(accp_ref)
        cnt_ref[...] = jnp.zeros_like(cnt_ref)

    h = _combine(i, acc_ref[...], den_ref[...], b_ref)
    seg = batch_ref[...]
    gid = lax.broadcasted_iota(jnp.int32, (ROW_BLK, G), 1)
    onehot = (seg == gid).astype(jnp.float32)
    accp_ref[...] += lax.dot_general(onehot, h, (((0,), (0,)), ((), ())),
                                     preferred_element_type=jnp.float32)
    cnt_ref[...] += lax.dot_general(onehot, jnp.ones_like(h),
                                    (((0,), (0,)), ((), ())),
                                    preferred_element_type=jnp.float32)

    @pl.when(i == N_BLKS - 1)
    def _():
        pooled = accp_ref[...] / jnp.maximum(cnt_ref[...], 1.0)
        out_ref[...] = jnp.tanh(
            lax.dot_general(pooled, wl_ref[...], (((1,), (1,)), ((), ())),
                            preferred_element_type=jnp.float32) + bl_ref[...])


def _tc_combine_pool(acc, den, b, batch_pad, Wl, bl):
    return pl.pallas_call(
        _combine_pool_kernel,
        grid=(N_BLKS,),
        in_specs=[
            pl.BlockSpec((NC, ROW_BLK, D), lambda i: (0, i, 0)),
            pl.BlockSpec((NC, ROW_BLK, 1), lambda i: (0, i, 0)),
            pl.BlockSpec((1, D), lambda i: (0, 0)),
            pl.BlockSpec((ROW_BLK, 1), lambda i: (i, 0)),
            pl.BlockSpec((D, D), lambda i: (0, 0)),
            pl.BlockSpec((1, D), lambda i: (0, 0)),
        ],
        out_specs=pl.BlockSpec((G, D), lambda i: (0, 0)),
        out_shape=jax.ShapeDtypeStruct((G, D), jnp.float32),
        scratch_shapes=[
            pltpu.VMEM((G, D), jnp.float32),
            pltpu.VMEM((G, D), jnp.float32),
        ],
    )(acc, den, b.reshape(1, D), batch_pad, Wl, bl.reshape(1, D))


def _split_acc(out):
    den = out[:, DEN_BASE:DEN_BASE + DEN_ROWS, :].reshape(NC, DEN_ROWS * D, 1)
    return out, den[:, :NPAD]


def kernel(x, edge_index, batch, data2, emb, W0, as0, ad0, b0, W1, as1, ad1,
           b1, Wl, bl):
    del data2
    x32 = x.astype(jnp.int32)
    xpad = jnp.concatenate([x32, jnp.zeros((NPAD - N,), jnp.int32)])
    h0 = _sc_gather_rows(emb, xpad)

    loop = jnp.arange(N, dtype=jnp.int32)
    src = jnp.concatenate(
        [edge_index[0].astype(jnp.int32), loop,
         jnp.zeros((EXTRA,), jnp.int32)]).reshape(NW, NCHUNK, CH)
    dst = jnp.concatenate(
        [edge_index[1].astype(jnp.int32), loop,
         jnp.full((EXTRA,), N, jnp.int32)]).reshape(NW, NCHUNK, CH)
    sd = jnp.stack([src, dst], axis=2)

    hw, asnadn = _tc_project(h0, W0, as0, ad0)
    acc0, den0 = _split_acc(_sc_edge(hw, asnadn[:, 0], asnadn[:, 1], sd))

    hw1, asnadn1 = _tc_combine_project(acc0, den0, b0, W1, as1, ad1)
    acc1, den1 = _split_acc(_sc_edge(hw1, asnadn1[:, 0], asnadn1[:, 1], sd))

    batch_pad = jnp.concatenate(
        [batch.astype(jnp.int32), jnp.full((NPAD - N,), G, jnp.int32)])
    return _tc_combine_pool(acc1, den1, b1, batch_pad.reshape(NPAD, 1), Wl, bl)


# speedup vs baseline: 27.0397x; 1.1387x over previous
import dataclasses
import functools

import jax
import jax.numpy as jnp
from jax import lax
from jax.experimental import pallas as pl
from jax.experimental.pallas import tpu as pltpu
from jax.experimental.pallas import tpu_sc as plsc

N = 10000
D = 128
G = 64
E = 320000

NC = 2
NS = 16
NW = NC * NS

NPAD = 10240
ROW_BLK = 1280
N_BLKS = NPAD // ROW_BLK
XPT = NPAD // NW

CH = 64
NCHUNK = 162
EPAD = NW * NCHUNK * CH
EXTRA = EPAD - E - N

DEN_BASE = 10112
DEN_ROWS = 80
APAD = 10240
RPT = APAD // NS

_vector_mesh = plsc.VectorSubcoreMesh(core_axis_name="c", subcore_axis_name="s")

_sc_params = pltpu.CompilerParams()
if "needs_layout_passes" in pltpu.CompilerParams.__dataclass_fields__:
    _sc_params = dataclasses.replace(_sc_params, needs_layout_passes=False)


def _sc_gather_rows(table, idx):
    @functools.partial(
        pl.kernel,
        mesh=_vector_mesh,
        out_type=jax.ShapeDtypeStruct((NPAD, D), jnp.float32),
        scratch_types=[
            pltpu.VMEM((XPT,), jnp.int32),
            pltpu.VMEM((XPT, D), jnp.float32),
            pltpu.SemaphoreType.DMA,
        ],
    )
    def k(table_hbm, idx_hbm, out_hbm, idx_v, rows_v, sem):
        wid = lax.axis_index("s") * NC + lax.axis_index("c")
        base = wid * XPT
        pltpu.sync_copy(idx_hbm.at[pl.ds(base, XPT)], idx_v)
        pltpu.async_copy(table_hbm.at[idx_v], rows_v, sem).wait()
        pltpu.sync_copy(rows_v, out_hbm.at[pl.ds(base, XPT)])

    return k(table, idx)


def _sc_edge_kernel(hw_hbm, asn_hbm, adn_hbm, sd_hbm, out_hbm,
                    asn_t, adn_t, sd2_t, dsc_t, ee_t, rows2_t, den_t,
                    den_rows_t, acc, gsem0, gsem1, ssem0, ssem1, dsem0, dsem1):
    gsem = (gsem0, gsem1)
    ssem = (ssem0, ssem1)
    dsem = (dsem0, dsem1)
    c = lax.axis_index("c")
    s = lax.axis_index("s")
    wid = s * NC + c

    pltpu.sync_copy(asn_hbm, asn_t)
    pltpu.sync_copy(adn_hbm, adn_t)

    zv = jnp.zeros((16,), jnp.float32)
    zvi = jnp.zeros((16,), jnp.int32)
    i16 = lax.broadcasted_iota(jnp.int32, (16,), 0)

    @pl.loop(0, CH)
    def _(r):
        @pl.loop(0, D, step=16)
        def _(k2):
            rows2_t[0, r, pl.ds(k2, 16)] = zv
            rows2_t[1, r, pl.ds(k2, 16)] = zv

    @pl.loop(0, CH, step=16)
    def _(g):
        dsc_t[0, pl.ds(g, 16)] = zvi
        dsc_t[1, pl.ds(g, 16)] = zvi

    @pl.loop(0, DEN_ROWS)
    def _(r):
        @pl.loop(0, D, step=16)
        def _(k2):
            den_t[r, pl.ds(k2, 16)] = zv

    @pl.loop(0, DEN_ROWS, step=16)
    def _(k2):
        den_rows_t[pl.ds(k2, 16)] = DEN_BASE + k2 + i16

    @pl.loop(0, RPT, step=CH)
    def _(q):
        pltpu.sync_copy(rows2_t.at[0], acc.at[pl.ds(s * RPT + q, CH)])

    plsc.subcore_barrier()

    pltpu.async_copy(rows2_t.at[0], acc.at[dsc_t.at[0]], ssem[0], add=True)
    pltpu.async_copy(rows2_t.at[1], acc.at[dsc_t.at[1]], ssem[1], add=True)
    pltpu.async_copy(sd_hbm.at[wid, 0], sd2_t.at[0], dsem[0])
    pltpu.async_copy(sd_hbm.at[wid, 1], sd2_t.at[1], dsem[1])

    @pl.loop(0, NCHUNK, step=2)
    def _(j):
        for b in range(2):
            i = j + b
            pltpu.make_async_copy(hw_hbm.at[pl.ds(0, CH)], rows2_t.at[b],
                                  ssem[b]).wait()
            pltpu.make_async_copy(sd_hbm.at[wid, 0], sd2_t.at[b],
                                  dsem[b]).wait()
            gcopy = pltpu.async_copy(hw_hbm.at[sd2_t.at[b, 0]],
                                     rows2_t.at[b], gsem[b])

            @plsc.parallel_loop(0, CH, step=16, unroll=2)
            def _(g):
                sv = sd2_t[b, 0, pl.ds(g, 16)]
                dv = sd2_t[b, 1, pl.ds(g, 16)]
                e = (plsc.load_gather(asn_t, [sv])
                     + plsc.load_gather(adn_t, [dv]))
                e = jnp.where(e >= 0.0, e, 0.2 * e)
                ee = jnp.exp(e)
                ee_t[pl.ds(g, 16)] = ee
                plsc.addupdate_scatter(
                    den_t, [lax.shift_right_logical(dv, 7),
                            lax.bitwise_and(dv, 127)], ee)
                dsc_t[b, pl.ds(g, 16)] = dv

            gcopy.wait()
            pltpu.async_copy(sd_hbm.at[wid, jnp.minimum(i + 2, NCHUNK - 1)],
                             sd2_t.at[b], dsem[b])

            @plsc.parallel_loop(0, CH, step=1, unroll=4)
            def _(r):
                eev = plsc.load_gather(ee_t, [zvi + r])
                for k2 in range(D // 16):
                    rows2_t[b, r, pl.ds(k2 * 16, 16)] = (
                        rows2_t[b, r, pl.ds(k2 * 16, 16)] * eev)

            pltpu.async_copy(rows2_t.at[b], acc.at[dsc_t.at[b]], ssem[b],
                             add=True)

    for b in range(2):
        pltpu.make_async_copy(hw_hbm.at[pl.ds(0, CH)], rows2_t.at[b],
                              ssem[b]).wait()
        pltpu.make_async_copy(sd_hbm.at[wid, 0], sd2_t.at[b], dsem[b]).wait()

    pltpu.sync_copy(den_t, acc.at[den_rows_t], add=True)

    plsc.subcore_barrier()

    @pl.loop(0, RPT, step=2 * CH)
    def _(q):
        r0 = s * RPT + q
        pltpu.sync_copy(acc.at[pl.ds(r0, 2 * CH)],
                        out_hbm.at[c, pl.ds(r0, 2 * CH)])


def _sc_edge(hw, asn, adn, sd):
    @functools.partial(
        pl.kernel,
        mesh=_vector_mesh,
        compiler_params=_sc_params,
        out_type=jax.ShapeDtypeStruct((NC, APAD, D), jnp.float32),
        scratch_types=[
            pltpu.VMEM((NPAD,), jnp.float32),
            pltpu.VMEM((NPAD,), jnp.float32),
            pltpu.VMEM((2, 2, CH), jnp.int32),
            pltpu.VMEM((2, CH), jnp.int32),
            pltpu.VMEM((CH,), jnp.float32),
            pltpu.VMEM((2, CH, D), jnp.float32),
            pltpu.VMEM((DEN_ROWS, D), jnp.float32),
            pltpu.VMEM((DEN_ROWS,), jnp.int32),
            pltpu.VMEM_SHARED((APAD, D), jnp.float32),
            pltpu.SemaphoreType.DMA,
            pltpu.SemaphoreType.DMA,
            pltpu.SemaphoreType.DMA,
            pltpu.SemaphoreType.DMA,
            pltpu.SemaphoreType.DMA,
            pltpu.SemaphoreType.DMA,
        ],
    )
    def k(*refs):
        _sc_edge_kernel(*refs)

    return k(hw, asn, adn, sd)


def _row_mask(i):
    row = i * ROW_BLK + lax.broadcasted_iota(jnp.int32, (ROW_BLK, 1), 0)
    return row < N


def _project_kernel(h_ref, w_ref, asd_ref, hw_ref, asnadn_ref):
    hw = lax.dot_general(h_ref[...], w_ref[...], (((1,), (1,)), ((), ())),
                         preferred_element_type=jnp.float32)
    hw_ref[...] = hw
    asnadn_ref[...] = lax.dot_general(hw, asd_ref[...], (((1,), (1,)), ((), ())),
                                      preferred_element_type=jnp.float32)


def _tc_project(h, W, a_s, a_d):
    asd = jnp.stack([a_s, a_d])
    return pl.pallas_call(
        _project_kernel,
        grid=(N_BLKS,),
        in_specs=[
            pl.BlockSpec((ROW_BLK, D), lambda i: (i, 0)),
            pl.BlockSpec((D, D), lambda i: (0, 0)),
            pl.BlockSpec((2, D), lambda i: (0, 0)),
        ],
        out_specs=[
            pl.BlockSpec((ROW_BLK, D), lambda i: (i, 0)),
            pl.BlockSpec((ROW_BLK, 2), lambda i: (i, 0)),
        ],
        out_shape=[
            jax.ShapeDtypeStruct((NPAD, D), jnp.float32),
            jax.ShapeDtypeStruct((NPAD, 2), jnp.float32),
        ],
    )(h, W, asd)


def _combine(i, acc_arr, den_arr, b_ref):
    num = acc_arr[0] + acc_arr[1]
    den = den_arr[0] + den_arr[1]
    h = jnp.tanh(num / (den + 1e-16) + b_ref[...])
    return jnp.where(_row_mask(i), h, 0.0)


def _combine_project_kernel(acc_ref, den_ref, b_ref, w_ref, asd_ref,
                            hw_ref, asnadn_ref):
    h = _combine(pl.program_id(0), acc_ref[...], den_ref[...], b_ref)
    hw = lax.dot_general(h, w_ref[...], (((1,), (1,)), ((), ())),
                         preferred_element_type=jnp.float32)
    hw_ref[...] = hw
    asnadn_ref[...] = lax.dot_general(hw, asd_ref[...], (((1,), (1,)), ((), ())),
                                      preferred_element_type=jnp.float32)


def _tc_combine_project(acc, den, b, W, a_s, a_d):
    asd = jnp.stack([a_s, a_d])
    return pl.pallas_call(
        _combine_project_kernel,
        grid=(N_BLKS,),
        in_specs=[
            pl.BlockSpec((NC, ROW_BLK, D), lambda i: (0, i, 0)),
            pl.BlockSpec((NC, ROW_BLK, 1), lambda i: (0, i, 0)),
            pl.BlockSpec((1, D), lambda i: (0, 0)),
            pl.BlockSpec((D, D), lambda i: (0, 0)),
            pl.BlockSpec((2, D), lambda i: (0, 0)),
        ],
        out_specs=[
            pl.BlockSpec((ROW_BLK, D), lambda i: (i, 0)),
            pl.BlockSpec((ROW_BLK, 2), lambda i: (i, 0)),
        ],
        out_shape=[
            jax.ShapeDtypeStruct((NPAD, D), jnp.float32),
            jax.ShapeDtypeStruct((NPAD, 2), jnp.float32),
        ],
    )(acc, den, b.reshape(1, D), W, asd)


def _combine_pool_kernel(acc_ref, den_ref, b_ref, batch_ref, wl_ref, bl_ref,
                         out_ref, accp_ref, cnt_ref):
    i = pl.program_id(0)

    @pl.when(i == 0)
    def _():
        accp_ref[...] = jnp.zeros_like(accp_ref)
        cnt_ref[...] = jnp.zeros_like(cnt_ref)

    h = _combine(i, acc_ref[...], den_ref[...], b_ref)
    seg = batch_ref[...]
    gid = lax.broadcasted_iota(jnp.int32, (ROW_BLK, G), 1)
    onehot = (seg == gid).astype(jnp.float32)
    accp_ref[...] += lax.dot_general(onehot, h, (((0,), (0,)), ((), ())),
                                     preferred_element_type=jnp.float32)
    cnt_ref[...] += lax.dot_general(onehot, jnp.ones_like(h),
                                    (((0,), (0,)), ((), ())),
                                    preferred_element_type=jnp.float32)

    @pl.when(i == N_BLKS - 1)
    def _():
        pooled = accp_ref[...] / jnp.maximum(cnt_ref[...], 1.0)
        out_ref[...] = jnp.tanh(
            lax.dot_general(pooled, wl_ref[...], (((1,), (1,)), ((), ())),
                            preferred_element_type=jnp.float32) + bl_ref[...])


def _tc_combine_pool(acc, den, b, batch_pad, Wl, bl):
    return pl.pallas_call(
        _combine_pool_kernel,
        grid=(N_BLKS,),
        in_specs=[
            pl.BlockSpec((NC, ROW_BLK, D), lambda i: (0, i, 0)),
            pl.BlockSpec((NC, ROW_BLK, 1), lambda i: (0, i, 0)),
            pl.BlockSpec((1, D), lambda i: (0, 0)),
            pl.BlockSpec((ROW_BLK, 1), lambda i: (i, 0)),
            pl.BlockSpec((D, D), lambda i: (0, 0)),
            pl.BlockSpec((1, D), lambda i: (0, 0)),
        ],
        out_specs=pl.BlockSpec((G, D), lambda i: (0, 0)),
        out_shape=jax.ShapeDtypeStruct((G, D), jnp.float32),
        scratch_shapes=[
            pltpu.VMEM((G, D), jnp.float32),
            pltpu.VMEM((G, D), jnp.float32),
        ],
    )(acc, den, b.reshape(1, D), batch_pad, Wl, bl.reshape(1, D))


def _split_acc(out):
    den = out[:, DEN_BASE:DEN_BASE + DEN_ROWS, :].reshape(NC, DEN_ROWS * D, 1)
    return out, den[:, :NPAD]


def kernel(x, edge_index, batch, data2, emb, W0, as0, ad0, b0, W1, as1, ad1,
           b1, Wl, bl):
    del data2
    x32 = x.astype(jnp.int32)
    xpad = jnp.concatenate([x32, jnp.zeros((NPAD - N,), jnp.int32)])
    h0 = _sc_gather_rows(emb, xpad)

    loop = jnp.arange(N, dtype=jnp.int32)
    src = jnp.concatenate(
        [edge_index[0].astype(jnp.int32), loop,
         jnp.zeros((EXTRA,), jnp.int32)]).reshape(NW, NCHUNK, CH)
    dst = jnp.concatenate(
        [edge_index[1].astype(jnp.int32), loop,
         jnp.full((EXTRA,), N, jnp.int32)]).reshape(NW, NCHUNK, CH)
    sd = jnp.stack([src, dst], axis=2)

    hw, asnadn = _tc_project(h0, W0, as0, ad0)
    acc0, den0 = _split_acc(_sc_edge(hw, asnadn[:, 0], asnadn[:, 1], sd))

    hw1, asnadn1 = _tc_combine_project(acc0, den0, b0, W1, as1, ad1)
    acc1, den1 = _split_acc(_sc_edge(hw1, asnadn1[:, 0], asnadn1[:, 1], sd))

    batch_pad = jnp.concatenate(
        [batch.astype(jnp.int32), jnp.full((NPAD - N,), G, jnp.int32)])
    return _tc_combine_pool(acc1, den1, b1, batch_pad.reshape(NPAD, 1), Wl, bl)


# speedup vs baseline: 27.9686x; 1.0344x over previous
import dataclasses
import functools

import jax
import jax.numpy as jnp
from jax import lax
from jax.experimental import pallas as pl
from jax.experimental.pallas import tpu as pltpu
from jax.experimental.pallas import tpu_sc as plsc

N = 10000
D = 128
G = 64
E = 320000

NC = 2
NS = 16
NW = NC * NS

NPAD = 10240
ROW_BLK = 1280
N_BLKS = NPAD // ROW_BLK
XPT = NPAD // NW

CH = 32
NCHUNK = 324
EPAD = NW * NCHUNK * CH
EXTRA = EPAD - E - N

DEN_BASE = 10112
DEN_ROWS = 80
APAD = 10240
RPT = APAD // NS

_vector_mesh = plsc.VectorSubcoreMesh(core_axis_name="c", subcore_axis_name="s")

_sc_params = pltpu.CompilerParams()
if "needs_layout_passes" in pltpu.CompilerParams.__dataclass_fields__:
    _sc_params = dataclasses.replace(_sc_params, needs_layout_passes=False)


def _sc_gather_rows(table, idx):
    @functools.partial(
        pl.kernel,
        mesh=_vector_mesh,
        out_type=jax.ShapeDtypeStruct((NPAD, D), jnp.float32),
        scratch_types=[
            pltpu.VMEM((XPT,), jnp.int32),
            pltpu.VMEM((XPT, D), jnp.float32),
            pltpu.SemaphoreType.DMA,
        ],
    )
    def k(table_hbm, idx_hbm, out_hbm, idx_v, rows_v, sem):
        wid = lax.axis_index("s") * NC + lax.axis_index("c")
        base = wid * XPT
        pltpu.sync_copy(idx_hbm.at[pl.ds(base, XPT)], idx_v)
        pltpu.async_copy(table_hbm.at[idx_v], rows_v, sem).wait()
        pltpu.sync_copy(rows_v, out_hbm.at[pl.ds(base, XPT)])

    return k(table, idx)


def _sc_edge_kernel(hw_hbm, asn_hbm, adn_hbm, sd_hbm, out_hbm,
                    asn_t, adn_t, sd2_t, dsc_t, ee_t, rows4_t, den_t,
                    den_rows_t, acc, *sems):
    gsem = sems[0:4]
    ssem = sems[4:8]
    dsem = sems[8:12]
    c = lax.axis_index("c")
    s = lax.axis_index("s")
    wid = s * NC + c

    pltpu.sync_copy(asn_hbm, asn_t)
    pltpu.sync_copy(adn_hbm, adn_t)

    zv = jnp.zeros((16,), jnp.float32)
    zvi = jnp.zeros((16,), jnp.int32)
    i16 = lax.broadcasted_iota(jnp.int32, (16,), 0)

    @pl.loop(0, CH)
    def _(r):
        for b in range(4):
            @pl.loop(0, D, step=16)
            def _(k2):
                rows4_t[b, r, pl.ds(k2, 16)] = zv

    @pl.loop(0, CH, step=16)
    def _(g):
        for b in range(4):
            dsc_t[b, pl.ds(g, 16)] = zvi

    @pl.loop(0, DEN_ROWS)
    def _(r):
        @pl.loop(0, D, step=16)
        def _(k2):
            den_t[r, pl.ds(k2, 16)] = zv

    @pl.loop(0, DEN_ROWS, step=16)
    def _(k2):
        den_rows_t[pl.ds(k2, 16)] = DEN_BASE + k2 + i16

    @pl.loop(0, RPT, step=CH)
    def _(q):
        pltpu.sync_copy(rows4_t.at[0], acc.at[pl.ds(s * RPT + q, CH)])

    plsc.subcore_barrier()

    for b in range(1, 4):
        pltpu.async_copy(rows4_t.at[b], acc.at[dsc_t.at[b]], ssem[b],
                         add=True)
    pltpu.sync_copy(sd_hbm.at[wid, 0], sd2_t.at[0])
    pltpu.async_copy(sd_hbm.at[wid, 1], sd2_t.at[1], dsem[1])
    pltpu.async_copy(hw_hbm.at[sd2_t.at[0, 0]], rows4_t.at[0], gsem[0])

    @pl.loop(0, NCHUNK, step=4)
    def _(j):
        for b in range(4):
            bn = (b + 1) % 4
            b2 = (b + 2) % 4
            i = j + b
            pltpu.make_async_copy(sd_hbm.at[wid, 0], sd2_t.at[bn],
                                  dsem[bn]).wait()
            pltpu.make_async_copy(hw_hbm.at[pl.ds(0, CH)], rows4_t.at[bn],
                                  ssem[bn]).wait()
            pltpu.async_copy(hw_hbm.at[sd2_t.at[bn, 0]], rows4_t.at[bn],
                             gsem[bn])

            @plsc.parallel_loop(0, CH, step=16, unroll=2)
            def _(g):
                sv = sd2_t[b, 0, pl.ds(g, 16)]
                dv = sd2_t[b, 1, pl.ds(g, 16)]
                e = (plsc.load_gather(asn_t, [sv])
                     + plsc.load_gather(adn_t, [dv]))
                e = jnp.where(e >= 0.0, e, 0.2 * e)
                ee = jnp.exp(e)
                ee_t[pl.ds(g, 16)] = ee
                plsc.addupdate_scatter(
                    den_t, [lax.shift_right_logical(dv, 7),
                            lax.bitwise_and(dv, 127)], ee)
                dsc_t[b, pl.ds(g, 16)] = dv

            pltpu.make_async_copy(hw_hbm.at[pl.ds(0, CH)], rows4_t.at[b],
                                  gsem[b]).wait()
            pltpu.async_copy(sd_hbm.at[wid, jnp.minimum(i + 2, NCHUNK - 1)],
                             sd2_t.at[b2], dsem[b2])

            @plsc.parallel_loop(0, CH, step=1, unroll=4)
            def _(r):
                eev = plsc.load_gather(ee_t, [zvi + r])
                for k2 in range(D // 16):
                    rows4_t[b, r, pl.ds(k2 * 16, 16)] = (
                        rows4_t[b, r, pl.ds(k2 * 16, 16)] * eev)

            pltpu.async_copy(rows4_t.at[b], acc.at[dsc_t.at[b]], ssem[b],
                             add=True)

    for b in range(1, 4):
        pltpu.make_async_copy(hw_hbm.at[pl.ds(0, CH)], rows4_t.at[b],
                              ssem[b]).wait()
    pltpu.make_async_copy(hw_hbm.at[pl.ds(0, CH)], rows4_t.at[0],
                          gsem[0]).wait()
    pltpu.make_async_copy(sd_hbm.at[wid, 0], sd2_t.at[1], dsem[1]).wait()

    pltpu.sync_copy(den_t, acc.at[den_rows_t], add=True)

    plsc.subcore_barrier()

    @pl.loop(0, RPT, step=2 * CH)
    def _(q):
        r0 = s * RPT + q
        pltpu.sync_copy(acc.at[pl.ds(r0, 2 * CH)],
                        out_hbm.at[c, pl.ds(r0, 2 * CH)])


def _sc_edge(hw, asn, adn, sd):
    @functools.partial(
        pl.kernel,
        mesh=_vector_mesh,
        compiler_params=_sc_params,
        out_type=jax.ShapeDtypeStruct((NC, APAD, D), jnp.float32),
        scratch_types=[
            pltpu.VMEM((NPAD,), jnp.float32),
            pltpu.VMEM((NPAD,), jnp.float32),
            pltpu.VMEM((4, 2, CH), jnp.int32),
            pltpu.VMEM((4, CH), jnp.int32),
            pltpu.VMEM((CH,), jnp.float32),
            pltpu.VMEM((4, CH, D), jnp.float32),
            pltpu.VMEM((DEN_ROWS, D), jnp.float32),
            pltpu.VMEM((DEN_ROWS,), jnp.int32),
            pltpu.VMEM_SHARED((APAD, D), jnp.float32),
        ] + [pltpu.SemaphoreType.DMA] * 12,
    )
    def k(*refs):
        _sc_edge_kernel(*refs)

    return k(hw, asn, adn, sd)


def _row_mask(i):
    row = i * ROW_BLK + lax.broadcasted_iota(jnp.int32, (ROW_BLK, 1), 0)
    return row < N


def _project_kernel(h_ref, w_ref, asd_ref, hw_ref, asnadn_ref):
    hw = lax.dot_general(h_ref[...], w_ref[...], (((1,), (1,)), ((), ())),
                         preferred_element_type=jnp.float32)
    hw_ref[...] = hw
    asnadn_ref[...] = lax.dot_general(hw, asd_ref[...], (((1,), (1,)), ((), ())),
                                      preferred_element_type=jnp.float32)


def _tc_project(h, W, a_s, a_d):
    asd = jnp.stack([a_s, a_d])
    return pl.pallas_call(
        _project_kernel,
        grid=(N_BLKS,),
        in_specs=[
            pl.BlockSpec((ROW_BLK, D), lambda i: (i, 0)),
            pl.BlockSpec((D, D), lambda i: (0, 0)),
            pl.BlockSpec((2, D), lambda i: (0, 0)),
        ],
        out_specs=[
            pl.BlockSpec((ROW_BLK, D), lambda i: (i, 0)),
            pl.BlockSpec((ROW_BLK, 2), lambda i: (i, 0)),
        ],
        out_shape=[
            jax.ShapeDtypeStruct((NPAD, D), jnp.float32),
            jax.ShapeDtypeStruct((NPAD, 2), jnp.float32),
        ],
    )(h, W, asd)


def _combine(i, acc_arr, den_arr, b_ref):
    num = acc_arr[0] + acc_arr[1]
    den = den_arr[0] + den_arr[1]
    h = jnp.tanh(num / (den + 1e-16) + b_ref[...])
    return jnp.where(_row_mask(i), h, 0.0)


def _combine_project_kernel(acc_ref, den_ref, b_ref, w_ref, asd_ref,
                            hw_ref, asnadn_ref):
    h = _combine(pl.program_id(0), acc_ref[...], den_ref[...], b_ref)
    hw = lax.dot_general(h, w_ref[...], (((1,), (1,)), ((), ())),
                         preferred_element_type=jnp.float32)
    hw_ref[...] = hw
    asnadn_ref[...] = lax.dot_general(hw, asd_ref[...], (((1,), (1,)), ((), ())),
                                      preferred_element_type=jnp.float32)


def _tc_combine_project(acc, den, b, W, a_s, a_d):
    asd = jnp.stack([a_s, a_d])
    return pl.pallas_call(
        _combine_project_kernel,
        grid=(N_BLKS,),
        in_specs=[
            pl.BlockSpec((NC, ROW_BLK, D), lambda i: (0, i, 0)),
            pl.BlockSpec((NC, ROW_BLK, 1), lambda i: (0, i, 0)),
            pl.BlockSpec((1, D), lambda i: (0, 0)),
            pl.BlockSpec((D, D), lambda i: (0, 0)),
            pl.BlockSpec((2, D), lambda i: (0, 0)),
        ],
        out_specs=[
            pl.BlockSpec((ROW_BLK, D), lambda i: (i, 0)),
            pl.BlockSpec((ROW_BLK, 2), lambda i: (i, 0)),
        ],
        out_shape=[
            jax.ShapeDtypeStruct((NPAD, D), jnp.float32),
            jax.ShapeDtypeStruct((NPAD, 2), jnp.float32),
        ],
    )(acc, den, b.reshape(1, D), W, asd)


def _combine_pool_kernel(acc_ref, den_ref, b_ref, batch_ref, wl_ref, bl_ref,
                         out_ref, accp_ref, cnt_ref):
    i = pl.program_id(0)

    @pl.when(i == 0)
    def _():
        accp_ref[...] = jnp.zeros_like(accp_ref)
        cnt_ref[...] = jnp.zeros_like(cnt_ref)

    h = _combine(i, acc_ref[...], den_ref[...], b_ref)
    seg = batch_ref[...]
    gid = lax.broadcasted_iota(jnp.int32, (ROW_BLK, G), 1)
    onehot = (seg == gid).astype(jnp.float32)
    accp_ref[...] += lax.dot_general(onehot, h, (((0,), (0,)), ((), ())),
                                     preferred_element_type=jnp.float32)
    cnt_ref[...] += lax.dot_general(onehot, jnp.ones_like(h),
                                    (((0,), (0,)), ((), ())),
                                    preferred_element_type=jnp.float32)

    @pl.when(i == N_BLKS - 1)
    def _():
        pooled = accp_ref[...] / jnp.maximum(cnt_ref[...], 1.0)
        out_ref[...] = jnp.tanh(
            lax.dot_general(pooled, wl_ref[...], (((1,), (1,)), ((), ())),
                            preferred_element_type=jnp.float32) + bl_ref[...])


def _tc_combine_pool(acc, den, b, batch_pad, Wl, bl):
    return pl.pallas_call(
        _combine_pool_kernel,
        grid=(N_BLKS,),
        in_specs=[
            pl.BlockSpec((NC, ROW_BLK, D), lambda i: (0, i, 0)),
            pl.BlockSpec((NC, ROW_BLK, 1), lambda i: (0, i, 0)),
            pl.BlockSpec((1, D), lambda i: (0, 0)),
            pl.BlockSpec((ROW_BLK, 1), lambda i: (i, 0)),
            pl.BlockSpec((D, D), lambda i: (0, 0)),
            pl.BlockSpec((1, D), lambda i: (0, 0)),
        ],
        out_specs=pl.BlockSpec((G, D), lambda i: (0, 0)),
        out_shape=jax.ShapeDtypeStruct((G, D), jnp.float32),
        scratch_shapes=[
            pltpu.VMEM((G, D), jnp.float32),
            pltpu.VMEM((G, D), jnp.float32),
        ],
    )(acc, den, b.reshape(1, D), batch_pad, Wl, bl.reshape(1, D))


def _split_acc(out):
    den = out[:, DEN_BASE:DEN_BASE + DEN_ROWS, :].reshape(NC, DEN_ROWS * D, 1)
    return out, den[:, :NPAD]


def kernel(x, edge_index, batch, data2, emb, W0, as0, ad0, b0, W1, as1, ad1,
           b1, Wl, bl):
    del data2
    x32 = x.astype(jnp.int32)
    xpad = jnp.concatenate([x32, jnp.zeros((NPAD - N,), jnp.int32)])
    h0 = _sc_gather_rows(emb, xpad)

    loop = jnp.arange(N, dtype=jnp.int32)
    src = jnp.concatenate(
        [edge_index[0].astype(jnp.int32), loop,
         jnp.zeros((EXTRA,), jnp.int32)]).reshape(NW, NCHUNK, CH)
    dst = jnp.concatenate(
        [edge_index[1].astype(jnp.int32), loop,
         jnp.full((EXTRA,), N, jnp.int32)]).reshape(NW, NCHUNK, CH)
    sd = jnp.stack([src, dst], axis=2)

    hw, asnadn = _tc_project(h0, W0, as0, ad0)
    acc0, den0 = _split_acc(_sc_edge(hw, asnadn[:, 0], asnadn[:, 1], sd))

    hw1, asnadn1 = _tc_combine_project(acc0, den0, b0, W1, as1, ad1)
    acc1, den1 = _split_acc(_sc_edge(hw1, asnadn1[:, 0], asnadn1[:, 1], sd))

    batch_pad = jnp.concatenate(
        [batch.astype(jnp.int32), jnp.full((NPAD - N,), G, jnp.int32)])
    return _tc_combine_pool(acc1, den1, b1, batch_pad.reshape(NPAD, 1), Wl, bl)
